# baseline (speedup 1.0000x reference)
import functools

import jax
import jax.numpy as jnp
from jax import lax
from jax.experimental import pallas as pl
from jax.experimental.pallas import tpu as pltpu
from jax.experimental.pallas import tpu_sc as plsc

_NPOINT = 512
_NSAMPLES = [16, 32]
_KMAX = 32
_CPAD = 128
_EPS = 1e-5


def _coord_tournament(v, x, y, z):
    w = v.shape[1] // 2
    while w >= 128:
        take = v[:, w:] > v[:, :w]
        v = jnp.where(take, v[:, w:], v[:, :w])
        x = jnp.where(take, x[:, w:], x[:, :w])
        y = jnp.where(take, y[:, w:], y[:, :w])
        z = jnp.where(take, z[:, w:], z[:, :w])
        w //= 2
    B, W = v.shape
    f = jnp.argmax(v, axis=1).astype(jnp.int32)[:, None]
    onehot = lax.broadcasted_iota(jnp.int32, (B, W), 1) == f
    cx = jnp.sum(jnp.where(onehot, x, 0.0), axis=1, keepdims=True)
    cy = jnp.sum(jnp.where(onehot, y, 0.0), axis=1, keepdims=True)
    cz = jnp.sum(jnp.where(onehot, z, 0.0), axis=1, keepdims=True)
    return cx, cy, cz


def _fps_body(px_ref, py_ref, pz_ref, f0_ref, ox_ref, oy_ref, oz_ref):
    px = px_ref[...]
    py = py_ref[...]
    pz = pz_ref[...]
    B, N = px.shape
    S = ox_ref.shape[1]
    lane = lax.broadcasted_iota(jnp.int32, (B, N), 1)
    col = lax.broadcasted_iota(jnp.int32, (B, S), 1)

    onehot = lane == f0_ref[...]
    cx0 = jnp.sum(jnp.where(onehot, px, 0.0), axis=1, keepdims=True)
    cy0 = jnp.sum(jnp.where(onehot, py, 0.0), axis=1, keepdims=True)
    cz0 = jnp.sum(jnp.where(onehot, pz, 0.0), axis=1, keepdims=True)

    def step(i, carry):
        dist_min, cx, cy, cz, ax, ay, az = carry
        ax = jnp.where(col == i, cx, ax)
        ay = jnp.where(col == i, cy, ay)
        az = jnp.where(col == i, cz, az)
        dx = px - cx
        dy = py - cy
        dz = pz - cz
        d = dx * dx + dy * dy + dz * dz
        dist_min = jnp.minimum(dist_min, d)
        cx, cy, cz = _coord_tournament(dist_min, px, py, pz)
        return dist_min, cx, cy, cz, ax, ay, az

    dist0 = jnp.full((B, N), 1e10, dtype=jnp.float32)
    zeros = jnp.zeros((B, S), dtype=jnp.float32)
    _, _, _, _, ax, ay, az = lax.fori_loop(
        0, S, step, (dist0, cx0, cy0, cz0, zeros, zeros, zeros)
    )
    ox_ref[...] = ax
    oy_ref[...] = ay
    oz_ref[...] = az


def _fps(px, py, pz, f0):
    B, N = px.shape
    out = jax.ShapeDtypeStruct((B, _NPOINT), jnp.float32)
    return pl.pallas_call(
        _fps_body,
        out_shape=(out, out, out),
    )(px, py, pz, f0)


_QBLK = 256


def _knn_body(px_ref, py_ref, pz_ref, qx_ref, qy_ref, qz_ref, idx_ref):
    b = pl.program_id(0)
    px = px_ref[0, 0, :][None, :]
    py = py_ref[0, 0, :][None, :]
    pz = pz_ref[0, 0, :][None, :]
    qx = qx_ref[0, 0, :][:, None]
    qy = qy_ref[0, 0, :][:, None]
    qz = qz_ref[0, 0, :][:, None]
    dx = qx - px
    dy = qy - py
    dz = qz - pz
    d2 = dx * dx + dy * dy + dz * dz
    N = d2.shape[1]
    lane = lax.broadcasted_iota(jnp.int32, (_QBLK, N), 1)
    base = b * N
    for k in range(_KMAX):
        am = jnp.argmin(d2, axis=1).astype(jnp.int32)
        idx_ref[0, pl.ds(k, 1), :] = (am + base)[None, :]
        d2 = jnp.where(lane == am[:, None], jnp.inf, d2)


def _knn(px, py, pz, nx, ny, nz):
    B, N = px.shape
    S = nx.shape[1]
    grid = (B, S // _QBLK)
    p_spec = pl.BlockSpec((1, 1, N), lambda b, q: (b, 0, 0))
    q_spec = pl.BlockSpec((1, 1, _QBLK), lambda b, q: (b, 0, q))
    idx_spec = pl.BlockSpec((1, _KMAX, _QBLK), lambda b, q: (b, 0, q))
    return pl.pallas_call(
        _knn_body,
        grid=grid,
        in_specs=[p_spec, p_spec, p_spec, q_spec, q_spec, q_spec],
        out_specs=idx_spec,
        out_shape=jax.ShapeDtypeStruct((B, _KMAX, S), jnp.int32),
    )(px[:, None, :], py[:, None, :], pz[:, None, :],
      nx[:, None, :], ny[:, None, :], nz[:, None, :])


_GCHUNK = 256


def _gather_rows(table, idx):
    R = idx.shape[0]
    NW = 32
    per_w = R // NW
    n_chunks = per_w // _GCHUNK
    mesh = plsc.VectorSubcoreMesh(core_axis_name="c", subcore_axis_name="s")

    @functools.partial(
        pl.kernel,
        out_type=jax.ShapeDtypeStruct((R, _CPAD), jnp.float32),
        mesh=mesh,
        scratch_types=[
            pltpu.VMEM((_GCHUNK,), jnp.int32),
            pltpu.VMEM((_GCHUNK,), jnp.int32),
            pltpu.VMEM((_GCHUNK, _CPAD), jnp.float32),
            pltpu.VMEM((_GCHUNK, _CPAD), jnp.float32),
            pltpu.SemaphoreType.DMA,
            pltpu.SemaphoreType.DMA,
        ],
    )
    def gather_kernel(
        table_hbm, idx_hbm, out_hbm, idx_v0, idx_v1, rows_v0, rows_v1, sem0, sem1
    ):
        wid = lax.axis_index("s") * 2 + lax.axis_index("c")
        base = wid * per_w
        idx_vs = (idx_v0, idx_v1)
        rows_vs = (rows_v0, rows_v1)
        sems = (sem0, sem1)
        handles = [None, None]
        pltpu.sync_copy(idx_hbm.at[pl.ds(base, _GCHUNK)], idx_v0)
        handles[0] = pltpu.async_copy(table_hbm.at[idx_v0], rows_v0, sem0)
        for c in range(n_chunks):
            p = c % 2
            q = (c + 1) % 2
            if c + 1 < n_chunks:
                off = base + (c + 1) * _GCHUNK
                pltpu.sync_copy(idx_hbm.at[pl.ds(off, _GCHUNK)], idx_vs[q])
                handles[q] = pltpu.async_copy(
                    table_hbm.at[idx_vs[q]], rows_vs[q], sems[q]
                )
            handles[p].wait()
            pltpu.sync_copy(rows_vs[p], out_hbm.at[pl.ds(base + c * _GCHUNK, _GCHUNK)])

    return gather_kernel(table, idx)


_GBLK = 256


_PBLK = 2048


def _proj_body(x3_ref, feat_ref, wf_ref, w3_ref, o_ref):
    y = jnp.dot(feat_ref[...], wf_ref[...], preferred_element_type=jnp.float32)
    x3 = x3_ref[...]
    w3 = w3_ref[...]
    o_ref[...] = (
        y
        + x3[:, 0:1] * w3[0:1, :]
        + x3[:, 1:2] * w3[1:2, :]
        + x3[:, 2:3] * w3[2:3, :]
    )


def _proj(x3, feat, wf, w3):
    V = x3.shape[0]
    grid = (V // _PBLK,)
    return pl.pallas_call(
        _proj_body,
        grid=grid,
        in_specs=[
            pl.BlockSpec((_PBLK, 3), lambda i: (i, 0)),
            pl.BlockSpec((_PBLK, feat.shape[1]), lambda i: (i, 0)),
            pl.BlockSpec(wf.shape, lambda i: (0, 0)),
            pl.BlockSpec(w3.shape, lambda i: (0, 0)),
        ],
        out_specs=pl.BlockSpec((_PBLK, _CPAD), lambda i: (i, 0)),
        out_shape=jax.ShapeDtypeStruct((V, _CPAD), jnp.float32),
    )(x3, feat, wf, w3)


def _stats(y):
    s = jnp.sum(y, axis=0, keepdims=True)
    ss = jnp.sum(y * y, axis=0, keepdims=True)
    return jnp.concatenate([s, ss], axis=0)


def _mlp_l0_body(K0, g_ref, nxyz_ref, w3_ref, y0_ref, y1_ref, st0_ref, st1_ref):
    gb = pl.program_id(0)
    g = g_ref[...]
    nxyz = nxyz_ref[...]
    w3 = w3_ref[...]
    cp = (
        nxyz[:, 0:1] * w3[0:1, :]
        + nxyz[:, 1:2] * w3[1:2, :]
        + nxyz[:, 2:3] * w3[2:3, :]
    )
    y = g - cp[:, None, :]
    y0 = y[:, :K0, 0:64].reshape(_GBLK * K0, 64)
    y1 = y[:, :, 64:128].reshape(_GBLK * _KMAX, 64)

    @pl.when(gb == 0)
    def _():
        st0_ref[...] = jnp.zeros_like(st0_ref)
        st1_ref[...] = jnp.zeros_like(st1_ref)

    st0_ref[...] += _stats(y0)
    st1_ref[...] += _stats(y1)
    y0_ref[...] = y0
    y1_ref[...] = y1


def _mlp_l0(g4, nxyz, w3, K0):
    G = g4.shape[0]
    grid = (G // _GBLK,)
    st_shape = jax.ShapeDtypeStruct((2, 64), jnp.float32)
    return pl.pallas_call(
        functools.partial(_mlp_l0_body, K0),
        grid=grid,
        in_specs=[
            pl.BlockSpec((_GBLK, _KMAX, _CPAD), lambda i: (i, 0, 0)),
            pl.BlockSpec((_GBLK, 3), lambda i: (i, 0)),
            pl.BlockSpec(w3.shape, lambda i: (0, 0)),
        ],
        out_specs=[
            pl.BlockSpec((_GBLK * K0, 64), lambda i: (i, 0)),
            pl.BlockSpec((_GBLK * _KMAX, 64), lambda i: (i, 0)),
            pl.BlockSpec((2, 64), lambda i: (0, 0)),
            pl.BlockSpec((2, 64), lambda i: (0, 0)),
        ],
        out_shape=[
            jax.ShapeDtypeStruct((G * K0, 64), jnp.float32),
            jax.ShapeDtypeStruct((G * _KMAX, 64), jnp.float32),
            st_shape,
            st_shape,
        ],
    )(g4, nxyz, w3)


def _norm_relu(y, st, r):
    mu = st[0:1, :] / r
    var = st[1:2, :] / r - mu * mu
    inv = lax.rsqrt(var + _EPS)
    return jnp.maximum((y - mu) * inv, 0.0)


def _mlp_mid2_body(
    r0, r1,
    y0_ref, y1_ref, st0_ref, st1_ref, w0_ref, w1_ref,
    o0_ref, o1_ref, ost0_ref, ost1_ref,
):
    gb = pl.program_id(0)
    x0 = _norm_relu(y0_ref[...], st0_ref[...], r0)
    y0 = jnp.dot(x0, w0_ref[...], preferred_element_type=jnp.float32)
    x1 = _norm_relu(y1_ref[...], st1_ref[...], r1)
    y1 = jnp.dot(x1, w1_ref[...], preferred_element_type=jnp.float32)

    @pl.when(gb == 0)
    def _():
        ost0_ref[...] = jnp.zeros_like(ost0_ref)
        ost1_ref[...] = jnp.zeros_like(ost1_ref)

    ost0_ref[...] += _stats(y0)
    ost1_ref[...] += _stats(y1)
    o0_ref[...] = y0
    o1_ref[...] = y1


def _mlp_mid2(y0, y1, st0, st1, w0, w1):
    R0, Cin0 = y0.shape
    R1, Cin1 = y1.shape
    rb0 = _GBLK * _NSAMPLES[0]
    rb1 = _GBLK * _NSAMPLES[1]
    grid = (R1 // rb1,)
    return pl.pallas_call(
        functools.partial(_mlp_mid2_body, float(R0), float(R1)),
        grid=grid,
        in_specs=[
            pl.BlockSpec((rb0, Cin0), lambda i: (i, 0)),
            pl.BlockSpec((rb1, Cin1), lambda i: (i, 0)),
            pl.BlockSpec((2, Cin0), lambda i: (0, 0)),
            pl.BlockSpec((2, Cin1), lambda i: (0, 0)),
            pl.BlockSpec(w0.shape, lambda i: (0, 0)),
            pl.BlockSpec(w1.shape, lambda i: (0, 0)),
        ],
        out_specs=[
            pl.BlockSpec((rb0, w0.shape[1]), lambda i: (i, 0)),
            pl.BlockSpec((rb1, w1.shape[1]), lambda i: (i, 0)),
            pl.BlockSpec((2, w0.shape[1]), lambda i: (0, 0)),
            pl.BlockSpec((2, w1.shape[1]), lambda i: (0, 0)),
        ],
        out_shape=[
            jax.ShapeDtypeStruct((R0, w0.shape[1]), jnp.float32),
            jax.ShapeDtypeStruct((R1, w1.shape[1]), jnp.float32),
            jax.ShapeDtypeStruct((2, w0.shape[1]), jnp.float32),
            jax.ShapeDtypeStruct((2, w1.shape[1]), jnp.float32),
        ],
    )(y0, y1, st0, st1, w0, w1)


def _mlp_final2_body(r0, r1, y0_ref, y1_ref, st0_ref, st1_ref, o_ref):
    x0 = _norm_relu(y0_ref[...], st0_ref[...], r0)
    x1 = _norm_relu(y1_ref[...], st1_ref[...], r1)
    m0 = jnp.max(x0.reshape(_GBLK, _NSAMPLES[0], 128), axis=1)
    m1 = jnp.max(x1.reshape(_GBLK, _NSAMPLES[1], 128), axis=1)
    o_ref[...] = jnp.concatenate([m0, m1], axis=-1)


def _mlp_final2(y0, y1, st0, st1):
    R0, _ = y0.shape
    R1, _ = y1.shape
    rb0 = _GBLK * _NSAMPLES[0]
    rb1 = _GBLK * _NSAMPLES[1]
    grid = (R1 // rb1,)
    return pl.pallas_call(
        functools.partial(_mlp_final2_body, float(R0), float(R1)),
        grid=grid,
        in_specs=[
            pl.BlockSpec((rb0, 128), lambda i: (i, 0)),
            pl.BlockSpec((rb1, 128), lambda i: (i, 0)),
            pl.BlockSpec((2, 128), lambda i: (0, 0)),
            pl.BlockSpec((2, 128), lambda i: (0, 0)),
        ],
        out_specs=pl.BlockSpec((_GBLK, 256), lambda i: (i, 0)),
        out_shape=jax.ShapeDtypeStruct((R1 // _NSAMPLES[1], 256), jnp.float32),
    )(y0, y1, st0, st1)


def kernel(xyz, features, W0_0, W0_1, W0_2, W1_0, W1_1, W1_2):
    B, N, _ = xyz.shape
    C = features.shape[2]
    S = _NPOINT

    px = xyz[:, :, 0]
    py = xyz[:, :, 1]
    pz = xyz[:, :, 2]
    f0 = jax.random.randint(jax.random.key(42), (B,), 0, N).astype(jnp.int32)

    nx, ny, nz = _fps(px, py, pz, f0[:, None])
    idx = _knn(px, py, pz, nx, ny, nz)
    idx_flat = idx.transpose(0, 2, 1).reshape(-1)

    wf = jnp.concatenate([W0_0.T[3:], W1_0.T[3:]], axis=1)
    w3 = jnp.concatenate([W0_0.T[:3], W1_0.T[:3]], axis=1)
    proj = _proj(xyz.reshape(B * N, 3), features.reshape(B * N, C), wf, w3)
    g = _gather_rows(proj, idx_flat)
    g4 = g.reshape(B * S, _KMAX, _CPAD)
    nxyz = jnp.stack([nx, ny, nz], axis=-1).reshape(B * S, 3)

    y0, y1, st0, st1 = _mlp_l0(g4, nxyz, w3, _NSAMPLES[0])
    y0, y1, st0, st1 = _mlp_mid2(y0, y1, st0, st1, W0_1.T, W1_1.T)
    y0, y1, st0, st1 = _mlp_mid2(y0, y1, st0, st1, W0_2.T, W1_2.T)
    fused = _mlp_final2(y0, y1, st0, st1).reshape(B, S, 256)

    new_xyz = jnp.stack([nx, ny, nz], axis=-1)
    return new_xyz, fused


# speedup vs baseline: 1.0721x; 1.0721x over previous
import functools

import jax
import jax.numpy as jnp
from jax import lax
from jax.experimental import pallas as pl
from jax.experimental.pallas import tpu as pltpu
from jax.experimental.pallas import tpu_sc as plsc

_NPOINT = 512
_NSAMPLES = [16, 32]
_KMAX = 32
_CPAD = 128
_EPS = 1e-5


def _coord_tournament(v, x, y, z):
    w = v.shape[1] // 2
    while w >= 128:
        take = v[:, w:] > v[:, :w]
        v = jnp.where(take, v[:, w:], v[:, :w])
        x = jnp.where(take, x[:, w:], x[:, :w])
        y = jnp.where(take, y[:, w:], y[:, :w])
        z = jnp.where(take, z[:, w:], z[:, :w])
        w //= 2
    B, W = v.shape
    f = jnp.argmax(v, axis=1).astype(jnp.int32)[:, None]
    onehot = lax.broadcasted_iota(jnp.int32, (B, W), 1) == f
    cx = jnp.sum(jnp.where(onehot, x, 0.0), axis=1, keepdims=True)
    cy = jnp.sum(jnp.where(onehot, y, 0.0), axis=1, keepdims=True)
    cz = jnp.sum(jnp.where(onehot, z, 0.0), axis=1, keepdims=True)
    return cx, cy, cz


def _fps_body(px_ref, py_ref, pz_ref, f0_ref, ox_ref, oy_ref, oz_ref):
    px = px_ref[...]
    py = py_ref[...]
    pz = pz_ref[...]
    B, N = px.shape
    S = ox_ref.shape[1]
    lane = lax.broadcasted_iota(jnp.int32, (B, N), 1)
    col = lax.broadcasted_iota(jnp.int32, (B, S), 1)

    onehot = lane == f0_ref[...]
    cx0 = jnp.sum(jnp.where(onehot, px, 0.0), axis=1, keepdims=True)
    cy0 = jnp.sum(jnp.where(onehot, py, 0.0), axis=1, keepdims=True)
    cz0 = jnp.sum(jnp.where(onehot, pz, 0.0), axis=1, keepdims=True)

    def step(i, carry):
        dist_min, cx, cy, cz, ax, ay, az = carry
        ax = jnp.where(col == i, cx, ax)
        ay = jnp.where(col == i, cy, ay)
        az = jnp.where(col == i, cz, az)
        dx = px - cx
        dy = py - cy
        dz = pz - cz
        d = dx * dx + dy * dy + dz * dz
        dist_min = jnp.minimum(dist_min, d)
        cx, cy, cz = _coord_tournament(dist_min, px, py, pz)
        return dist_min, cx, cy, cz, ax, ay, az

    dist0 = jnp.full((B, N), 1e10, dtype=jnp.float32)
    zeros = jnp.zeros((B, S), dtype=jnp.float32)
    _, _, _, _, ax, ay, az = lax.fori_loop(
        0, S, step, (dist0, cx0, cy0, cz0, zeros, zeros, zeros)
    )
    ox_ref[...] = ax
    oy_ref[...] = ay
    oz_ref[...] = az


def _fps(px, py, pz, f0):
    B, N = px.shape
    out = jax.ShapeDtypeStruct((B, _NPOINT), jnp.float32)
    return pl.pallas_call(
        _fps_body,
        out_shape=(out, out, out),
    )(px, py, pz, f0)


_QBLK = 128


def _knn_body(px_ref, py_ref, pz_ref, qx_ref, qy_ref, qz_ref, idx_ref):
    b = pl.program_id(0)
    px = px_ref[0, 0, :][None, :]
    py = py_ref[0, 0, :][None, :]
    pz = pz_ref[0, 0, :][None, :]
    qx = qx_ref[0, 0, :][:, None]
    qy = qy_ref[0, 0, :][:, None]
    qz = qz_ref[0, 0, :][:, None]
    dx = qx - px
    dy = qy - py
    dz = qz - pz
    d2 = dx * dx + dy * dy + dz * dz
    N = d2.shape[1]
    lane = lax.broadcasted_iota(jnp.int32, (_QBLK, N), 1)
    base = b * N
    for k in range(_KMAX):
        am = jnp.argmin(d2, axis=1).astype(jnp.int32)
        idx_ref[0, pl.ds(k, 1), :] = (am + base)[None, :]
        d2 = jnp.where(lane == am[:, None], jnp.inf, d2)


def _knn(px, py, pz, nx, ny, nz):
    B, N = px.shape
    S = nx.shape[1]
    grid = (B, S // _QBLK)
    p_spec = pl.BlockSpec((1, 1, N), lambda b, q: (b, 0, 0))
    q_spec = pl.BlockSpec((1, 1, _QBLK), lambda b, q: (b, 0, q))
    idx_spec = pl.BlockSpec((1, _KMAX, _QBLK), lambda b, q: (b, 0, q))
    return pl.pallas_call(
        _knn_body,
        grid=grid,
        in_specs=[p_spec, p_spec, p_spec, q_spec, q_spec, q_spec],
        out_specs=idx_spec,
        out_shape=jax.ShapeDtypeStruct((B, _KMAX, S), jnp.int32),
    )(px[:, None, :], py[:, None, :], pz[:, None, :],
      nx[:, None, :], ny[:, None, :], nz[:, None, :])


_GCHUNK = 256


def _gather_rows(table, idx):
    R = idx.shape[0]
    NW = 32
    per_w = R // NW
    n_chunks = per_w // _GCHUNK
    mesh = plsc.VectorSubcoreMesh(core_axis_name="c", subcore_axis_name="s")

    @functools.partial(
        pl.kernel,
        out_type=jax.ShapeDtypeStruct((R, _CPAD), jnp.float32),
        mesh=mesh,
        scratch_types=[
            pltpu.VMEM((_GCHUNK,), jnp.int32),
            pltpu.VMEM((_GCHUNK,), jnp.int32),
            pltpu.VMEM((_GCHUNK, _CPAD), jnp.float32),
            pltpu.VMEM((_GCHUNK, _CPAD), jnp.float32),
            pltpu.SemaphoreType.DMA,
            pltpu.SemaphoreType.DMA,
        ],
    )
    def gather_kernel(
        table_hbm, idx_hbm, out_hbm, idx_v0, idx_v1, rows_v0, rows_v1, sem0, sem1
    ):
        wid = lax.axis_index("s") * 2 + lax.axis_index("c")
        base = wid * per_w
        idx_vs = (idx_v0, idx_v1)
        rows_vs = (rows_v0, rows_v1)
        sems = (sem0, sem1)
        handles = [None, None]
        pltpu.sync_copy(idx_hbm.at[pl.ds(base, _GCHUNK)], idx_v0)
        handles[0] = pltpu.async_copy(table_hbm.at[idx_v0], rows_v0, sem0)
        for c in range(n_chunks):
            p = c % 2
            q = (c + 1) % 2
            if c + 1 < n_chunks:
                off = base + (c + 1) * _GCHUNK
                pltpu.sync_copy(idx_hbm.at[pl.ds(off, _GCHUNK)], idx_vs[q])
                handles[q] = pltpu.async_copy(
                    table_hbm.at[idx_vs[q]], rows_vs[q], sems[q]
                )
            handles[p].wait()
            pltpu.sync_copy(rows_vs[p], out_hbm.at[pl.ds(base + c * _GCHUNK, _GCHUNK)])

    return gather_kernel(table, idx)


_GBLK = 256


_PBLK = 2048


def _proj_body(x3_ref, feat_ref, wf_ref, w3_ref, o_ref):
    y = jnp.dot(feat_ref[...], wf_ref[...], preferred_element_type=jnp.float32)
    x3 = x3_ref[...]
    w3 = w3_ref[...]
    o_ref[...] = (
        y
        + x3[:, 0:1] * w3[0:1, :]
        + x3[:, 1:2] * w3[1:2, :]
        + x3[:, 2:3] * w3[2:3, :]
    )


def _proj(x3, feat, wf, w3):
    V = x3.shape[0]
    grid = (V // _PBLK,)
    return pl.pallas_call(
        _proj_body,
        grid=grid,
        in_specs=[
            pl.BlockSpec((_PBLK, 3), lambda i: (i, 0)),
            pl.BlockSpec((_PBLK, feat.shape[1]), lambda i: (i, 0)),
            pl.BlockSpec(wf.shape, lambda i: (0, 0)),
            pl.BlockSpec(w3.shape, lambda i: (0, 0)),
        ],
        out_specs=pl.BlockSpec((_PBLK, _CPAD), lambda i: (i, 0)),
        out_shape=jax.ShapeDtypeStruct((V, _CPAD), jnp.float32),
    )(x3, feat, wf, w3)


def _stats(y):
    s = jnp.sum(y, axis=0, keepdims=True)
    ss = jnp.sum(y * y, axis=0, keepdims=True)
    return jnp.concatenate([s, ss], axis=0)


def _mlp_l0_body(K0, g_ref, nxyz_ref, w3_ref, y0_ref, y1_ref, st0_ref, st1_ref):
    gb = pl.program_id(0)
    g = g_ref[...]
    nxyz = nxyz_ref[...]
    w3 = w3_ref[...]
    cp = (
        nxyz[:, 0:1] * w3[0:1, :]
        + nxyz[:, 1:2] * w3[1:2, :]
        + nxyz[:, 2:3] * w3[2:3, :]
    )
    y = g - cp[:, None, :]
    y0 = y[:, :K0, 0:64].reshape(_GBLK * K0, 64)
    y1 = y[:, :, 64:128].reshape(_GBLK * _KMAX, 64)

    @pl.when(gb == 0)
    def _():
        st0_ref[...] = jnp.zeros_like(st0_ref)
        st1_ref[...] = jnp.zeros_like(st1_ref)

    st0_ref[...] += _stats(y0)
    st1_ref[...] += _stats(y1)
    y0_ref[...] = y0
    y1_ref[...] = y1


def _mlp_l0(g4, nxyz, w3, K0):
    G = g4.shape[0]
    grid = (G // _GBLK,)
    st_shape = jax.ShapeDtypeStruct((2, 64), jnp.float32)
    return pl.pallas_call(
        functools.partial(_mlp_l0_body, K0),
        grid=grid,
        in_specs=[
            pl.BlockSpec((_GBLK, _KMAX, _CPAD), lambda i: (i, 0, 0)),
            pl.BlockSpec((_GBLK, 3), lambda i: (i, 0)),
            pl.BlockSpec(w3.shape, lambda i: (0, 0)),
        ],
        out_specs=[
            pl.BlockSpec((_GBLK * K0, 64), lambda i: (i, 0)),
            pl.BlockSpec((_GBLK * _KMAX, 64), lambda i: (i, 0)),
            pl.BlockSpec((2, 64), lambda i: (0, 0)),
            pl.BlockSpec((2, 64), lambda i: (0, 0)),
        ],
        out_shape=[
            jax.ShapeDtypeStruct((G * K0, 64), jnp.float32),
            jax.ShapeDtypeStruct((G * _KMAX, 64), jnp.float32),
            st_shape,
            st_shape,
        ],
    )(g4, nxyz, w3)


def _norm_relu(y, st, r):
    mu = st[0:1, :] / r
    var = st[1:2, :] / r - mu * mu
    inv = lax.rsqrt(var + _EPS)
    return jnp.maximum((y - mu) * inv, 0.0)


def _mlp_mid2_body(
    r0, r1,
    y0_ref, y1_ref, st0_ref, st1_ref, w0_ref, w1_ref,
    o0_ref, o1_ref, ost0_ref, ost1_ref,
):
    gb = pl.program_id(0)
    x0 = _norm_relu(y0_ref[...], st0_ref[...], r0)
    y0 = jnp.dot(x0, w0_ref[...], preferred_element_type=jnp.float32)
    x1 = _norm_relu(y1_ref[...], st1_ref[...], r1)
    y1 = jnp.dot(x1, w1_ref[...], preferred_element_type=jnp.float32)

    @pl.when(gb == 0)
    def _():
        ost0_ref[...] = jnp.zeros_like(ost0_ref)
        ost1_ref[...] = jnp.zeros_like(ost1_ref)

    ost0_ref[...] += _stats(y0)
    ost1_ref[...] += _stats(y1)
    o0_ref[...] = y0
    o1_ref[...] = y1


def _mlp_mid2(y0, y1, st0, st1, w0, w1):
    R0, Cin0 = y0.shape
    R1, Cin1 = y1.shape
    rb0 = _GBLK * _NSAMPLES[0]
    rb1 = _GBLK * _NSAMPLES[1]
    grid = (R1 // rb1,)
    return pl.pallas_call(
        functools.partial(_mlp_mid2_body, float(R0), float(R1)),
        grid=grid,
        in_specs=[
            pl.BlockSpec((rb0, Cin0), lambda i: (i, 0)),
            pl.BlockSpec((rb1, Cin1), lambda i: (i, 0)),
            pl.BlockSpec((2, Cin0), lambda i: (0, 0)),
            pl.BlockSpec((2, Cin1), lambda i: (0, 0)),
            pl.BlockSpec(w0.shape, lambda i: (0, 0)),
            pl.BlockSpec(w1.shape, lambda i: (0, 0)),
        ],
        out_specs=[
            pl.BlockSpec((rb0, w0.shape[1]), lambda i: (i, 0)),
            pl.BlockSpec((rb1, w1.shape[1]), lambda i: (i, 0)),
            pl.BlockSpec((2, w0.shape[1]), lambda i: (0, 0)),
            pl.BlockSpec((2, w1.shape[1]), lambda i: (0, 0)),
        ],
        out_shape=[
            jax.ShapeDtypeStruct((R0, w0.shape[1]), jnp.float32),
            jax.ShapeDtypeStruct((R1, w1.shape[1]), jnp.float32),
            jax.ShapeDtypeStruct((2, w0.shape[1]), jnp.float32),
            jax.ShapeDtypeStruct((2, w1.shape[1]), jnp.float32),
        ],
    )(y0, y1, st0, st1, w0, w1)


def _mlp_final2_body(r0, r1, y0_ref, y1_ref, st0_ref, st1_ref, o_ref):
    x0 = _norm_relu(y0_ref[...], st0_ref[...], r0)
    x1 = _norm_relu(y1_ref[...], st1_ref[...], r1)
    m0 = jnp.max(x0.reshape(_GBLK, _NSAMPLES[0], 128), axis=1)
    m1 = jnp.max(x1.reshape(_GBLK, _NSAMPLES[1], 128), axis=1)
    o_ref[...] = jnp.concatenate([m0, m1], axis=-1)


def _mlp_final2(y0, y1, st0, st1):
    R0, _ = y0.shape
    R1, _ = y1.shape
    rb0 = _GBLK * _NSAMPLES[0]
    rb1 = _GBLK * _NSAMPLES[1]
    grid = (R1 // rb1,)
    return pl.pallas_call(
        functools.partial(_mlp_final2_body, float(R0), float(R1)),
        grid=grid,
        in_specs=[
            pl.BlockSpec((rb0, 128), lambda i: (i, 0)),
            pl.BlockSpec((rb1, 128), lambda i: (i, 0)),
            pl.BlockSpec((2, 128), lambda i: (0, 0)),
            pl.BlockSpec((2, 128), lambda i: (0, 0)),
        ],
        out_specs=pl.BlockSpec((_GBLK, 256), lambda i: (i, 0)),
        out_shape=jax.ShapeDtypeStruct((R1 // _NSAMPLES[1], 256), jnp.float32),
    )(y0, y1, st0, st1)


def kernel(xyz, features, W0_0, W0_1, W0_2, W1_0, W1_1, W1_2):
    B, N, _ = xyz.shape
    C = features.shape[2]
    S = _NPOINT

    px = xyz[:, :, 0]
    py = xyz[:, :, 1]
    pz = xyz[:, :, 2]
    f0 = jax.random.randint(jax.random.key(42), (B,), 0, N).astype(jnp.int32)

    nx, ny, nz = _fps(px, py, pz, f0[:, None])
    idx = _knn(px, py, pz, nx, ny, nz)
    idx_flat = idx.transpose(0, 2, 1).reshape(-1)

    wf = jnp.concatenate([W0_0.T[3:], W1_0.T[3:]], axis=1)
    w3 = jnp.concatenate([W0_0.T[:3], W1_0.T[:3]], axis=1)
    proj = _proj(xyz.reshape(B * N, 3), features.reshape(B * N, C), wf, w3)
    g = _gather_rows(proj, idx_flat)
    g4 = g.reshape(B * S, _KMAX, _CPAD)
    nxyz = jnp.stack([nx, ny, nz], axis=-1).reshape(B * S, 3)

    y0, y1, st0, st1 = _mlp_l0(g4, nxyz, w3, _NSAMPLES[0])
    y0, y1, st0, st1 = _mlp_mid2(y0, y1, st0, st1, W0_1.T, W1_1.T)
    y0, y1, st0, st1 = _mlp_mid2(y0, y1, st0, st1, W0_2.T, W1_2.T)
    fused = _mlp_final2(y0, y1, st0, st1).reshape(B, S, 256)

    new_xyz = jnp.stack([nx, ny, nz], axis=-1)
    return new_xyz, fused
